# baseline (speedup 1.0000x reference)
import functools

import jax
import jax.numpy as jnp
from jax import lax
from jax.experimental import pallas as pl
from jax.experimental.pallas import tpu as pltpu
from jax.experimental.pallas import tpu_sc as plsc

VOCAB = 1000000
EMBED = 64
BATCH = 16384
NEG_K = 20

NC = 2
NS = 16
NW = NC * NS

B_W = BATCH // NW
N_W = BATCH * NEG_K // NW
CHUNK = 1024
NCHUNK = N_W // CHUNK

_mesh = plsc.VectorSubcoreMesh(
    core_axis_name="c", subcore_axis_name="s", num_cores=NC, num_subcores=NS
)


@functools.partial(
    pl.kernel,
    out_type=(
        jax.ShapeDtypeStruct((BATCH, EMBED), jnp.float32),
        jax.ShapeDtypeStruct((BATCH, EMBED), jnp.float32),
        jax.ShapeDtypeStruct((BATCH * NEG_K, EMBED), jnp.float32),
    ),
    mesh=_mesh,
    scratch_types=[
        pltpu.VMEM((B_W,), jnp.int32),
        pltpu.VMEM((B_W, EMBED), jnp.float32),
        pltpu.VMEM((CHUNK,), jnp.int32),
        pltpu.VMEM((CHUNK, EMBED), jnp.float32),
        pltpu.SemaphoreType.DMA,
    ],
    compiler_params=pltpu.CompilerParams(use_tc_tiling_on_sc=False),
)
def _sc_gather(center_hbm, pos_hbm, neg_hbm, in_hbm, out_hbm,
               o_center, o_pos, o_neg,
               idx_v, rows_v, idx_ch, rows_ch, sem):
    wid = lax.axis_index("s") * NC + lax.axis_index("c")
    base = wid * B_W

    pltpu.sync_copy(center_hbm.at[pl.ds(base, B_W)], idx_v)
    pltpu.async_copy(in_hbm.at[idx_v], rows_v, sem).wait()
    pltpu.sync_copy(rows_v, o_center.at[pl.ds(base, B_W)])

    pltpu.sync_copy(pos_hbm.at[pl.ds(base, B_W)], idx_v)
    pltpu.async_copy(out_hbm.at[idx_v], rows_v, sem).wait()
    pltpu.sync_copy(rows_v, o_pos.at[pl.ds(base, B_W)])

    nbase = wid * N_W
    for c in range(NCHUNK):
        off = nbase + c * CHUNK
        pltpu.sync_copy(neg_hbm.at[pl.ds(off, CHUNK)], idx_ch)
        pltpu.async_copy(out_hbm.at[idx_ch], rows_ch, sem).wait()
        pltpu.sync_copy(rows_ch, o_neg.at[pl.ds(off, CHUNK)])


@jax.jit
def kernel(center_word, pos_word, neg_word, in_embed, out_embed):
    neg_flat = neg_word.reshape(BATCH * NEG_K)
    emb, pos, neg = _sc_gather(center_word, pos_word, neg_flat,
                               in_embed, out_embed)
    return emb, pos, neg.reshape(BATCH, NEG_K, EMBED)


# speedup vs baseline: 1.0103x; 1.0103x over previous
import functools

import jax
import jax.numpy as jnp
from jax import lax
from jax.experimental import pallas as pl
from jax.experimental.pallas import tpu as pltpu
from jax.experimental.pallas import tpu_sc as plsc

VOCAB = 1000000
EMBED = 64
BATCH = 16384
NEG_K = 20

NC = 2
NS = 16
NW = NC * NS

B_W = BATCH // NW
N_W = BATCH * NEG_K // NW
CHUNK = 512
NNEG = N_W // CHUNK
NJOBS = 2 + NNEG
IDX_TOTAL = B_W + B_W + N_W

_mesh = plsc.VectorSubcoreMesh(
    core_axis_name="c", subcore_axis_name="s", num_cores=NC, num_subcores=NS
)


@functools.partial(
    pl.kernel,
    out_type=(
        jax.ShapeDtypeStruct((BATCH, EMBED), jnp.float32),
        jax.ShapeDtypeStruct((BATCH, EMBED), jnp.float32),
        jax.ShapeDtypeStruct((BATCH * NEG_K, EMBED), jnp.float32),
    ),
    mesh=_mesh,
    scratch_types=[
        pltpu.VMEM((IDX_TOTAL,), jnp.int32),
        pltpu.VMEM((CHUNK, EMBED), jnp.float32),
        pltpu.VMEM((CHUNK, EMBED), jnp.float32),
        pltpu.SemaphoreType.DMA,
        pltpu.SemaphoreType.DMA,
        pltpu.SemaphoreType.DMA,
        pltpu.SemaphoreType.DMA,
        pltpu.SemaphoreType.DMA,
    ],
    compiler_params=pltpu.CompilerParams(use_tc_tiling_on_sc=False),
)
def _sc_gather(center_hbm, pos_hbm, neg_hbm, in_hbm, out_hbm,
               o_center, o_pos, o_neg,
               idx_all, rows0, rows1, sem_i, sg0, sg1, sw0, sw1):
    wid = lax.axis_index("s") * NC + lax.axis_index("c")
    base = wid * B_W
    nbase = wid * N_W

    ci = pltpu.async_copy(center_hbm.at[pl.ds(base, B_W)],
                          idx_all.at[pl.ds(0, B_W)], sem_i)
    pi = pltpu.async_copy(pos_hbm.at[pl.ds(base, B_W)],
                          idx_all.at[pl.ds(B_W, B_W)], sem_i)
    ni = pltpu.async_copy(neg_hbm.at[pl.ds(nbase, N_W)],
                          idx_all.at[pl.ds(2 * B_W, N_W)], sem_i)
    ci.wait()
    pi.wait()
    ni.wait()

    jobs = [(in_hbm, 0, o_center, base), (out_hbm, B_W, o_pos, base)]
    for c in range(NNEG):
        jobs.append((out_hbm, 2 * B_W + c * CHUNK, o_neg, nbase + c * CHUNK))

    rows = (rows0, rows1)
    sg = (sg0, sg1)
    sw = (sw0, sw1)
    gdesc = [None] * NJOBS
    wdesc = [None] * NJOBS

    for j in range(NJOBS):
        b = j % 2
        if j >= 2:
            wdesc[j - 2].wait()
        table, ioff, _, _ = jobs[j]
        gdesc[j] = pltpu.async_copy(
            table.at[idx_all.at[pl.ds(ioff, CHUNK)]], rows[b], sg[b])
        if j >= 1:
            gdesc[j - 1].wait()
            _, _, oref, ooff = jobs[j - 1]
            wdesc[j - 1] = pltpu.async_copy(
                rows[(j - 1) % 2], oref.at[pl.ds(ooff, CHUNK)], sw[(j - 1) % 2])

    gdesc[NJOBS - 1].wait()
    _, _, oref, ooff = jobs[NJOBS - 1]
    wdesc[NJOBS - 1] = pltpu.async_copy(
        rows[(NJOBS - 1) % 2], oref.at[pl.ds(ooff, CHUNK)],
        sw[(NJOBS - 1) % 2])
    wdesc[NJOBS - 2].wait()
    wdesc[NJOBS - 1].wait()


@jax.jit
def kernel(center_word, pos_word, neg_word, in_embed, out_embed):
    neg_flat = neg_word.reshape(BATCH * NEG_K)
    emb, pos, neg = _sc_gather(center_word, pos_word, neg_flat,
                               in_embed, out_embed)
    return emb, pos, neg.reshape(BATCH, NEG_K, EMBED)


# speedup vs baseline: 1.0316x; 1.0211x over previous
import functools

import jax
import jax.numpy as jnp
from jax import lax
from jax.experimental import pallas as pl
from jax.experimental.pallas import tpu as pltpu
from jax.experimental.pallas import tpu_sc as plsc

VOCAB = 1000000
EMBED = 64
BATCH = 16384
NEG_K = 20

NC = 2
NS = 16
NW = NC * NS

B_W = BATCH // NW

_mesh = plsc.VectorSubcoreMesh(
    core_axis_name="c", subcore_axis_name="s", num_cores=NC, num_subcores=NS
)


@functools.partial(
    pl.kernel,
    out_type=(
        jax.ShapeDtypeStruct((BATCH, EMBED), jnp.float32),
        jax.ShapeDtypeStruct((BATCH, EMBED), jnp.float32),
        jax.ShapeDtypeStruct((NEG_K, BATCH, EMBED), jnp.float32),
    ),
    mesh=_mesh,
    scratch_types=[
        pltpu.VMEM((B_W,), jnp.int32),
        pltpu.VMEM((B_W,), jnp.int32),
        pltpu.VMEM((NEG_K, B_W), jnp.int32),
        pltpu.VMEM((B_W, EMBED), jnp.float32),
        pltpu.VMEM((B_W, EMBED), jnp.float32),
        pltpu.SemaphoreType.DMA,
        pltpu.SemaphoreType.DMA,
        pltpu.SemaphoreType.DMA,
    ],
    compiler_params=pltpu.CompilerParams(use_tc_tiling_on_sc=False),
)
def _sc_gather(center_hbm, pos_hbm, negt_hbm, in_hbm, out_hbm,
               o_center, o_pos, o_neg,
               idxc, idxp, idxn, bufa, bufb, semi, sema, semb):
    wid = lax.axis_index("s") * NC + lax.axis_index("c")
    base = pl.multiple_of(wid * B_W, B_W)

    di = pltpu.async_copy(center_hbm.at[pl.ds(base, B_W)], idxc, semi)
    dp = pltpu.async_copy(pos_hbm.at[pl.ds(base, B_W)], idxp, semi)
    dn = pltpu.async_copy(negt_hbm.at[:, pl.ds(base, B_W)], idxn, semi)
    di.wait()
    ga = pltpu.async_copy(in_hbm.at[idxc], bufa, sema)
    dp.wait()
    gb = pltpu.async_copy(out_hbm.at[idxp], bufb, semb)
    ga.wait()
    pltpu.sync_copy(bufa, o_center.at[pl.ds(base, B_W)])
    dn.wait()
    pltpu.async_copy(out_hbm.at[idxn.at[0]], bufa, sema)
    gb.wait()
    pltpu.sync_copy(bufb, o_pos.at[pl.ds(base, B_W)])
    pltpu.async_copy(out_hbm.at[idxn.at[1]], bufb, semb)

    def neg_pair(j, _):
        ka = j * 2
        pltpu.make_async_copy(out_hbm.at[pl.ds(0, B_W)], bufa, sema).wait()
        pltpu.sync_copy(bufa, o_neg.at[ka, pl.ds(base, B_W), :])
        pltpu.async_copy(out_hbm.at[idxn.at[ka + 2]], bufa, sema)
        pltpu.make_async_copy(out_hbm.at[pl.ds(0, B_W)], bufb, semb).wait()
        pltpu.sync_copy(bufb, o_neg.at[ka + 1, pl.ds(base, B_W), :])
        pltpu.async_copy(out_hbm.at[idxn.at[ka + 3]], bufb, semb)
        return ()
    lax.fori_loop(0, (NEG_K - 2) // 2, neg_pair, ())

    pltpu.make_async_copy(out_hbm.at[pl.ds(0, B_W)], bufa, sema).wait()
    pltpu.sync_copy(bufa, o_neg.at[NEG_K - 2, pl.ds(base, B_W), :])
    pltpu.make_async_copy(out_hbm.at[pl.ds(0, B_W)], bufb, semb).wait()
    pltpu.sync_copy(bufb, o_neg.at[NEG_K - 1, pl.ds(base, B_W), :])


@jax.jit
def kernel(center_word, pos_word, neg_word, in_embed, out_embed):
    emb, pos, neg = _sc_gather(center_word, pos_word, neg_word.T,
                               in_embed, out_embed)
    return emb, pos, jnp.transpose(neg, (1, 0, 2))
